# Initial kernel scaffold; baseline (speedup 1.0000x reference)
import jax
import jax.numpy as jnp
from jax.experimental import pallas as pl


def kernel(h_user, h_item, edge_index):
    raise NotImplementedError("write your pallas kernel here")


# speedup vs baseline: 2.6700x; 2.6700x over previous
import functools

import jax
import jax.numpy as jnp
from jax import lax
from jax.experimental import pallas as pl
from jax.experimental.pallas import tpu as pltpu
from jax.experimental.pallas import tpu_sc as plsc

D = 128
L = 16
NC = 2
NS = 16
NW = NC * NS
CH = 80


def _normalize_body(u_ref, i_ref, ou_ref, oi_ref):
    for x_ref, o_ref in ((u_ref, ou_ref), (i_ref, oi_ref)):
        x = x_ref[...]
        s = jnp.sum(x * x, axis=1, keepdims=True)
        n = jnp.sqrt(s)
        o_ref[...] = x / jnp.maximum(n, 1e-12)


def _normalize(h_user, h_item):
    n, d = h_user.shape
    blk = 1000
    return pl.pallas_call(
        _normalize_body,
        grid=(n // blk,),
        in_specs=[pl.BlockSpec((blk, d), lambda i: (i, 0))] * 2,
        out_specs=[pl.BlockSpec((blk, d), lambda i: (i, 0))] * 2,
        out_shape=[jax.ShapeDtypeStruct((n, d), jnp.float32)] * 2,
    )(h_user, h_item)


_GATHER_DNUMS = lax.GatherDimensionNumbers(
    offset_dims=(), collapsed_slice_dims=(0,), start_index_map=(0,))


def _vperm(x, idx):
    return lax.gather(x, idx[:, None], _GATHER_DNUMS, (1,),
                      mode=lax.GatherScatterMode.PROMISE_IN_BOUNDS)


def _sc_body(epw, u_hbm, i_hbm, src_hbm, dst_hbm, out_hbm,
             idx_u, idx_v, rows_u, rows_v, out_v, sem_u, sem_v):
    wid = lax.axis_index("s") * NC + lax.axis_index("c")
    base = wid * epw
    nchunk = epw // CH
    lane = lax.iota(jnp.int32, L)
    perm_idx = [lane ^ (1 << t) for t in range(4)]

    def chunk_body(c, carry):
        off = base + c * CH
        pltpu.sync_copy(src_hbm.at[pl.ds(off, CH)], idx_u)
        pltpu.sync_copy(dst_hbm.at[pl.ds(off, CH)], idx_v)
        cp_u = pltpu.async_copy(u_hbm.at[idx_u], rows_u, sem_u)
        cp_v = pltpu.async_copy(i_hbm.at[idx_v], rows_v, sem_v)
        cp_u.wait()
        cp_v.wait()

        def group_body(g, carry2):
            e0 = g * L
            outv = jnp.zeros((L,), jnp.float32)
            for j in range(L):
                e = e0 + j
                acc = rows_u[e, pl.ds(0, L)] * rows_v[e, pl.ds(0, L)]
                for k in range(1, D // L):
                    acc = acc + rows_u[e, pl.ds(k * L, L)] * rows_v[e, pl.ds(k * L, L)]
                for t in range(4):
                    acc = acc + _vperm(acc, perm_idx[t])
                outv = jnp.where(lane == j, acc, outv)
            out_v[pl.ds(e0, L)] = outv
            return carry2

        lax.fori_loop(0, CH // L, group_body, 0)
        pltpu.sync_copy(out_v, out_hbm.at[pl.ds(off, CH)])
        return carry

    lax.fori_loop(0, nchunk, chunk_body, 0)


def kernel(h_user, h_item, edge_index):
    e = edge_index.shape[1]
    epw = e // NW
    norm_u, norm_i = _normalize(h_user, h_item)
    src = edge_index[0].astype(jnp.int32)
    dst = edge_index[1].astype(jnp.int32)

    mesh = plsc.VectorSubcoreMesh(core_axis_name="c", subcore_axis_name="s")
    ratings = pl.kernel(
        functools.partial(_sc_body, epw),
        out_type=jax.ShapeDtypeStruct((e,), jnp.float32),
        mesh=mesh,
        scratch_types=[
            pltpu.VMEM((CH,), jnp.int32),
            pltpu.VMEM((CH,), jnp.int32),
            pltpu.VMEM((CH, D), jnp.float32),
            pltpu.VMEM((CH, D), jnp.float32),
            pltpu.VMEM((CH,), jnp.float32),
            pltpu.SemaphoreType.DMA,
            pltpu.SemaphoreType.DMA,
        ],
    )(norm_u, norm_i, src, dst)
    return ratings.reshape(e, 1)


# speedup vs baseline: 4.5895x; 1.7189x over previous
import functools

import jax
import jax.numpy as jnp
from jax import lax
from jax.experimental import pallas as pl
from jax.experimental.pallas import tpu as pltpu
from jax.experimental.pallas import tpu_sc as plsc

D = 128
L = 16
NC = 2
NS = 16
NW = NC * NS
CH = 80


def _normalize_body(u_ref, i_ref, ou_ref, oi_ref):
    for x_ref, o_ref in ((u_ref, ou_ref), (i_ref, oi_ref)):
        x = x_ref[...]
        s = jnp.sum(x * x, axis=1, keepdims=True)
        n = jnp.sqrt(s)
        o_ref[...] = x / jnp.maximum(n, 1e-12)


def _normalize(h_user, h_item):
    n, d = h_user.shape
    blk = 1000
    return pl.pallas_call(
        _normalize_body,
        grid=(n // blk,),
        in_specs=[pl.BlockSpec((blk, d), lambda i: (i, 0))] * 2,
        out_specs=[pl.BlockSpec((blk, d), lambda i: (i, 0))] * 2,
        out_shape=[jax.ShapeDtypeStruct((n, d), jnp.float32)] * 2,
    )(h_user, h_item)


_GATHER_DNUMS = lax.GatherDimensionNumbers(
    offset_dims=(), collapsed_slice_dims=(0,), start_index_map=(0,))


def _vperm(x, idx):
    return lax.gather(x, idx[:, None], _GATHER_DNUMS, (1,),
                      mode=lax.GatherScatterMode.PROMISE_IN_BOUNDS)


def _sc_body(epw, u_hbm, i_hbm, src_hbm, dst_hbm, out_hbm,
             idx_src, idx_dst, rows_u, rows_v, out_v, sems):
    wid = lax.axis_index("s") * NC + lax.axis_index("c")
    base = wid * epw
    nchunk = epw // CH
    lane = lax.iota(jnp.int32, L)
    perm_idx = [lane ^ (1 << t) for t in range(4)]

    pltpu.sync_copy(src_hbm.at[pl.ds(base, epw)], idx_src)
    pltpu.sync_copy(dst_hbm.at[pl.ds(base, epw)], idx_dst)

    def issue(b, c):
        pltpu.async_copy(u_hbm.at[idx_src.at[pl.ds(c * CH, CH)]],
                         rows_u.at[b], sems.at[b, 0])
        pltpu.async_copy(i_hbm.at[idx_dst.at[pl.ds(c * CH, CH)]],
                         rows_v.at[b], sems.at[b, 1])

    def wait(b, c):
        pltpu.make_async_copy(u_hbm.at[idx_src.at[pl.ds(c * CH, CH)]],
                              rows_u.at[b], sems.at[b, 0]).wait()
        pltpu.make_async_copy(i_hbm.at[idx_dst.at[pl.ds(c * CH, CH)]],
                              rows_v.at[b], sems.at[b, 1]).wait()

    def compute(b, c):
        ru = rows_u.at[b]
        rv = rows_v.at[b]

        def group_body(g, carry2):
            e0 = g * L
            outv = jnp.zeros((L,), jnp.float32)
            for j in range(L):
                e = e0 + j
                acc = ru[e, pl.ds(0, L)] * rv[e, pl.ds(0, L)]
                for k in range(1, D // L):
                    acc = acc + ru[e, pl.ds(k * L, L)] * rv[e, pl.ds(k * L, L)]
                for t in range(4):
                    acc = acc + _vperm(acc, perm_idx[t])
                outv = jnp.where(lane == j, acc, outv)
            out_v[pl.ds(c * CH + e0, L)] = outv
            return carry2

        lax.fori_loop(0, CH // L, group_body, 0)

    issue(0, 0)

    def pipe_body(g, carry):
        c0 = 2 * g
        issue(1, c0 + 1)
        wait(0, c0)
        compute(0, c0)

        @pl.when(c0 + 2 < nchunk)
        def _():
            issue(0, c0 + 2)

        wait(1, c0 + 1)
        compute(1, c0 + 1)
        return carry

    lax.fori_loop(0, nchunk // 2, pipe_body, 0)
    if nchunk % 2 == 1:
        wait(0, nchunk - 1)
        compute(0, nchunk - 1)

    pltpu.sync_copy(out_v, out_hbm.at[pl.ds(base, epw)])


def kernel(h_user, h_item, edge_index):
    e = edge_index.shape[1]
    epw = e // NW
    norm_u, norm_i = _normalize(h_user, h_item)
    src = edge_index[0].astype(jnp.int32)
    dst = edge_index[1].astype(jnp.int32)

    mesh = plsc.VectorSubcoreMesh(core_axis_name="c", subcore_axis_name="s")
    ratings = pl.kernel(
        functools.partial(_sc_body, epw),
        out_type=jax.ShapeDtypeStruct((e,), jnp.float32),
        mesh=mesh,
        scratch_types=[
            pltpu.VMEM((epw,), jnp.int32),
            pltpu.VMEM((epw,), jnp.int32),
            pltpu.VMEM((2, CH, D), jnp.float32),
            pltpu.VMEM((2, CH, D), jnp.float32),
            pltpu.VMEM((epw,), jnp.float32),
            pltpu.SemaphoreType.DMA((2, 2)),
        ],
    )(norm_u, norm_i, src, dst)
    return ratings.reshape(e, 1)


# speedup vs baseline: 6.0696x; 1.3225x over previous
import functools

import jax
import jax.numpy as jnp
from jax import lax
from jax.experimental import pallas as pl
from jax.experimental.pallas import tpu as pltpu
from jax.experimental.pallas import tpu_sc as plsc

D = 128
DW = 64
L = 16
NC = 2
NS = 16
NW = NC * NS
CH = 80


def _normalize_body(u_ref, i_ref, ou_ref, oi_ref):
    for x_ref, o_ref in ((u_ref, ou_ref), (i_ref, oi_ref)):
        x = x_ref[...]
        s = jnp.sum(x * x, axis=1, keepdims=True)
        n = jnp.sqrt(s)
        o_ref[...] = (x / jnp.maximum(n, 1e-12)).astype(jnp.bfloat16)


def _normalize(h_user, h_item):
    n, d = h_user.shape
    blk = 1000
    return pl.pallas_call(
        _normalize_body,
        grid=(n // blk,),
        in_specs=[pl.BlockSpec((blk, d), lambda i: (i, 0))] * 2,
        out_specs=[pl.BlockSpec((blk, d), lambda i: (i, 0))] * 2,
        out_shape=[jax.ShapeDtypeStruct((n, d), jnp.bfloat16)] * 2,
    )(h_user, h_item)


_MASK_HI = -65536

_GATHER_DNUMS = lax.GatherDimensionNumbers(
    offset_dims=(), collapsed_slice_dims=(0,), start_index_map=(0,))


def _vperm(x, idx):
    return lax.gather(x, idx[:, None], _GATHER_DNUMS, (1,),
                      mode=lax.GatherScatterMode.PROMISE_IN_BOUNDS)


def _sc_body(epw, u_hbm, i_hbm, src_hbm, dst_hbm, out_hbm,
             idx_src, idx_dst, rows_u, rows_v, out_v, sems):
    wid = lax.axis_index("s") * NC + lax.axis_index("c")
    base = wid * epw
    nchunk = epw // CH
    lane = lax.iota(jnp.int32, L)
    perm_idx = [lane ^ (1 << t) for t in range(4)]

    pltpu.sync_copy(src_hbm.at[pl.ds(base, epw)], idx_src)
    pltpu.sync_copy(dst_hbm.at[pl.ds(base, epw)], idx_dst)

    def issue(b, c):
        pltpu.async_copy(u_hbm.at[idx_src.at[pl.ds(c * CH, CH)]],
                         rows_u.at[b], sems.at[b, 0])
        pltpu.async_copy(i_hbm.at[idx_dst.at[pl.ds(c * CH, CH)]],
                         rows_v.at[b], sems.at[b, 1])

    def wait(b, c):
        pltpu.make_async_copy(u_hbm.at[idx_src.at[pl.ds(c * CH, CH)]],
                              rows_u.at[b], sems.at[b, 0]).wait()
        pltpu.make_async_copy(i_hbm.at[idx_dst.at[pl.ds(c * CH, CH)]],
                              rows_v.at[b], sems.at[b, 1]).wait()

    def compute(b, c):
        ru = rows_u.at[b]
        rv = rows_v.at[b]

        def group_body(g, carry2):
            e0 = g * L
            outv = jnp.zeros((L,), jnp.float32)
            for j in range(L):
                e = e0 + j
                acc = jnp.zeros((L,), jnp.float32)
                for k in range(DW // L):
                    ub = plsc.bitcast(ru[e, pl.ds(k * L, L)], jnp.bfloat16)
                    vb = plsc.bitcast(rv[e, pl.ds(k * L, L)], jnp.bfloat16)
                    u0, u1 = plsc.unpack(ub, format=plsc.PackFormat.INTERLEAVED)
                    v0, v1 = plsc.unpack(vb, format=plsc.PackFormat.INTERLEAVED)
                    acc = acc + u0 * v0 + u1 * v1
                for t in range(4):
                    acc = acc + _vperm(acc, perm_idx[t])
                outv = jnp.where(lane == j, acc, outv)
            out_v[pl.ds(c * CH + e0, L)] = outv
            return carry2

        lax.fori_loop(0, CH // L, group_body, 0)

    issue(0, 0)

    def pipe_body(g, carry):
        c0 = 2 * g
        issue(1, c0 + 1)
        wait(0, c0)
        compute(0, c0)

        @pl.when(c0 + 2 < nchunk)
        def _():
            issue(0, c0 + 2)

        wait(1, c0 + 1)
        compute(1, c0 + 1)
        return carry

    lax.fori_loop(0, nchunk // 2, pipe_body, 0)
    if nchunk % 2 == 1:
        wait(0, nchunk - 1)
        compute(0, nchunk - 1)

    pltpu.sync_copy(out_v, out_hbm.at[pl.ds(base, epw)])


def kernel(h_user, h_item, edge_index):
    e = edge_index.shape[1]
    epw = e // NW
    norm_u, norm_i = _normalize(h_user, h_item)
    n = norm_u.shape[0]
    pu = lax.bitcast_convert_type(norm_u.reshape(n, DW, 2), jnp.int32)
    pi = lax.bitcast_convert_type(norm_i.reshape(n, DW, 2), jnp.int32)
    src = edge_index[0].astype(jnp.int32)
    dst = edge_index[1].astype(jnp.int32)

    mesh = plsc.VectorSubcoreMesh(core_axis_name="c", subcore_axis_name="s")
    ratings = pl.kernel(
        functools.partial(_sc_body, epw),
        out_type=jax.ShapeDtypeStruct((e,), jnp.float32),
        mesh=mesh,
        compiler_params=pltpu.CompilerParams(
            needs_layout_passes=False, use_tc_tiling_on_sc=False),
        scratch_types=[
            pltpu.VMEM((epw,), jnp.int32),
            pltpu.VMEM((epw,), jnp.int32),
            pltpu.VMEM((2, CH, DW), jnp.int32),
            pltpu.VMEM((2, CH, DW), jnp.int32),
            pltpu.VMEM((epw,), jnp.float32),
            pltpu.SemaphoreType.DMA((2, 2)),
        ],
    )(pu, pi, src, dst)
    return ratings.reshape(e, 1)


# speedup vs baseline: 7.7546x; 1.2776x over previous
import functools

import jax
import jax.numpy as jnp
from jax import lax
from jax.experimental import pallas as pl
from jax.experimental.pallas import tpu as pltpu
from jax.experimental.pallas import tpu_sc as plsc

D = 128
DW = 64
L = 16
NC = 2
NS = 16
NW = NC * NS
CH = 80


def _normalize_body(u_ref, i_ref, ou_ref, oi_ref):
    for x_ref, o_ref in ((u_ref, ou_ref), (i_ref, oi_ref)):
        x = x_ref[...]
        s = jnp.sum(x * x, axis=1, keepdims=True)
        n = jnp.sqrt(s)
        o_ref[...] = (x / jnp.maximum(n, 1e-12)).astype(jnp.bfloat16)


def _normalize(h_user, h_item):
    n, d = h_user.shape
    blk = 1000
    return pl.pallas_call(
        _normalize_body,
        grid=(n // blk,),
        in_specs=[pl.BlockSpec((blk, d), lambda i: (i, 0))] * 2,
        out_specs=[pl.BlockSpec((blk, d), lambda i: (i, 0))] * 2,
        out_shape=[jax.ShapeDtypeStruct((n, d), jnp.bfloat16)] * 2,
    )(h_user, h_item)


_MASK_HI = -65536

_GATHER_DNUMS = lax.GatherDimensionNumbers(
    offset_dims=(), collapsed_slice_dims=(0,), start_index_map=(0,))


def _vperm(x, idx):
    return lax.gather(x, idx[:, None], _GATHER_DNUMS, (1,),
                      mode=lax.GatherScatterMode.PROMISE_IN_BOUNDS)


def _sc_body(epw, u_hbm, i_hbm, src_hbm, dst_hbm, out_hbm,
             idx_src, idx_dst, rows_u, rows_v, out_v, sems):
    wid = lax.axis_index("s") * NC + lax.axis_index("c")
    base = wid * epw
    nchunk = epw // CH
    lane = lax.iota(jnp.int32, L)
    perm_idx = [lane ^ (1 << t) for t in range(4)]

    pltpu.sync_copy(src_hbm.at[pl.ds(base, epw)], idx_src)
    pltpu.sync_copy(dst_hbm.at[pl.ds(base, epw)], idx_dst)

    def issue(b, c):
        pltpu.async_copy(u_hbm.at[idx_src.at[pl.ds(c * CH, CH)]],
                         rows_u.at[b], sems.at[b, 0])
        pltpu.async_copy(i_hbm.at[idx_dst.at[pl.ds(c * CH, CH)]],
                         rows_v.at[b], sems.at[b, 1])

    def wait(b, c):
        pltpu.make_async_copy(u_hbm.at[idx_src.at[pl.ds(c * CH, CH)]],
                              rows_u.at[b], sems.at[b, 0]).wait()
        pltpu.make_async_copy(i_hbm.at[idx_dst.at[pl.ds(c * CH, CH)]],
                              rows_v.at[b], sems.at[b, 1]).wait()

    def compute(b, c):
        ru = rows_u.at[b]
        rv = rows_v.at[b]

        def group_body(g, carry2):
            e0 = g * L
            outv = jnp.zeros((L,), jnp.float32)
            for j in range(L):
                e = e0 + j
                acc = None
                for k in range(DW // L):
                    ub = plsc.bitcast(ru[e, pl.ds(k * L, L)], jnp.bfloat16)
                    vb = plsc.bitcast(rv[e, pl.ds(k * L, L)], jnp.bfloat16)
                    p0, p1 = plsc.unpack(ub * vb, format=plsc.PackFormat.INTERLEAVED)
                    ps = p0 + p1
                    acc = ps if acc is None else acc + ps
                for t in range(4):
                    acc = acc + _vperm(acc, perm_idx[t])
                outv = jnp.where(lane == j, acc, outv)
            out_v[pl.ds(c * CH + e0, L)] = outv
            return carry2

        lax.fori_loop(0, CH // L, group_body, 0)

    issue(0, 0)

    def pipe_body(g, carry):
        c0 = 2 * g
        issue(1, c0 + 1)
        wait(0, c0)
        compute(0, c0)

        @pl.when(c0 + 2 < nchunk)
        def _():
            issue(0, c0 + 2)

        wait(1, c0 + 1)
        compute(1, c0 + 1)
        return carry

    lax.fori_loop(0, nchunk // 2, pipe_body, 0)
    if nchunk % 2 == 1:
        wait(0, nchunk - 1)
        compute(0, nchunk - 1)

    pltpu.sync_copy(out_v, out_hbm.at[pl.ds(base, epw)])


def kernel(h_user, h_item, edge_index):
    e = edge_index.shape[1]
    epw = e // NW
    norm_u, norm_i = _normalize(h_user, h_item)
    n = norm_u.shape[0]
    pu = lax.bitcast_convert_type(norm_u.reshape(n, DW, 2), jnp.int32)
    pi = lax.bitcast_convert_type(norm_i.reshape(n, DW, 2), jnp.int32)
    src = edge_index[0].astype(jnp.int32)
    dst = edge_index[1].astype(jnp.int32)

    mesh = plsc.VectorSubcoreMesh(core_axis_name="c", subcore_axis_name="s")
    ratings = pl.kernel(
        functools.partial(_sc_body, epw),
        out_type=jax.ShapeDtypeStruct((e,), jnp.float32),
        mesh=mesh,
        compiler_params=pltpu.CompilerParams(
            needs_layout_passes=False, use_tc_tiling_on_sc=False),
        scratch_types=[
            pltpu.VMEM((epw,), jnp.int32),
            pltpu.VMEM((epw,), jnp.int32),
            pltpu.VMEM((2, CH, DW), jnp.int32),
            pltpu.VMEM((2, CH, DW), jnp.int32),
            pltpu.VMEM((epw,), jnp.float32),
            pltpu.SemaphoreType.DMA((2, 2)),
        ],
    )(pu, pi, src, dst)
    return ratings.reshape(e, 1)
